# baseline (speedup 1.0000x reference)
import functools

import jax
import jax.numpy as jnp
from jax import lax
from jax.experimental import pallas as pl
from jax.experimental.pallas import tpu as pltpu
from jax.experimental.pallas import tpu_sc as plsc

L = 8
NT = 1024
D = 1024
D2 = D // 2
N = 4
S = 2048
ROWS = N * S
NC = 1
NS = 16
NW = NC * NS
RPW = ROWS // NW
B = 4
NB = RPW // B
LANES = 16
PLANES = 32


def _fire(w_hbm, idx_v, gath_v, sem, b, buf):
    for l in range(L):
        pltpu.async_copy(
            w_hbm.at[idx_v.at[l, pl.ds(b * B, B)]],
            gath_v.at[buf, l],
            sem,
        )


def _drain(dummy_hbm, gath_v, sem, buf):
    pltpu.make_async_copy(
        dummy_hbm,
        gath_v.at[buf],
        sem,
    ).wait()


def _accum(gath_v, outb_v, buf):
    gb = gath_v.bitcast(jnp.bfloat16)
    ob = outb_v.bitcast(jnp.bfloat16)
    for j in range(B):
        def cbody(c, _, j=j):
            o = pl.ds(pl.multiple_of(c * LANES, LANES), LANES)
            r = pl.ds(2 * j, 2)
            acc = gb[buf, 0, r, o]
            for l in range(1, L):
                acc = acc + gb[buf, l, r, o]
            ob[buf, r, o] = acc
            return 0
        lax.fori_loop(0, D2 // LANES, cbody, 0)


def _body(x_hbm, w_hbm, dummy_hbm, out_hbm, idx_v, gath_v, outb_v, sem0, sem1):
    cid = lax.axis_index("c")
    sid = lax.axis_index("s")
    wid = sid * NC + cid
    n = wid // (S // RPW)
    s0 = (wid % (S // RPW)) * RPW
    row0 = wid * RPW

    for l in range(L):
        pltpu.sync_copy(x_hbm.at[n, l, pl.ds(s0, RPW)], idx_v.at[l])

    def off_body(i, _):
        o = pl.ds(pl.multiple_of(i * LANES, LANES), LANES)
        for l in range(1, L):
            idx_v[l, o] = idx_v[l, o] + (l * NT)
        return 0
    lax.fori_loop(0, RPW // LANES, off_body, 0)

    def _store(b, buf):
        pltpu.sync_copy(outb_v.at[buf], out_hbm.at[pl.ds(row0 + b * B, B)])

    _fire(w_hbm, idx_v, gath_v, sem0, 0, 0)

    def outer(bb, _):
        b0 = 2 * bb
        b1 = 2 * bb + 1
        _fire(w_hbm, idx_v, gath_v, sem1, b1, 1)
        _drain(dummy_hbm, gath_v, sem0, 0)
        _accum(gath_v, outb_v, 0)
        _store(b0, 0)
        _fire(w_hbm, idx_v, gath_v, sem0, jnp.minimum(b1 + 1, NB - 1), 0)
        _drain(dummy_hbm, gath_v, sem1, 1)
        _accum(gath_v, outb_v, 1)
        _store(b1, 1)
        return 0

    lax.fori_loop(0, NB // 2, outer, 0)
    _drain(dummy_hbm, gath_v, sem0, 0)


_mek = functools.partial(
    pl.kernel,
    out_type=jax.ShapeDtypeStruct((ROWS, D2), jnp.int32),
    mesh=plsc.VectorSubcoreMesh(core_axis_name="c", subcore_axis_name="s", num_cores=1),
    scratch_types=[
        pltpu.VMEM((L, RPW), jnp.int32),
        pltpu.VMEM((2, L, B, D2), jnp.int32),
        pltpu.VMEM((2, B, D2), jnp.int32),
        pltpu.SemaphoreType.DMA,
        pltpu.SemaphoreType.DMA,
    ],
)(_body)


@jax.jit
def kernel(x, weight):
    x = x.astype(jnp.int32)
    u = jax.lax.bitcast_convert_type(weight.reshape(L * NT, D), jnp.uint32)
    rne = lambda v: (v + jnp.uint32(0x7FFF) + ((v >> 16) & jnp.uint32(1))) >> 16
    lo = rne(u[:, :D2])
    hi = rne(u[:, D2:])
    w_flat = jax.lax.bitcast_convert_type(lo | (hi << 16), jnp.int32)
    dummy = jnp.zeros((L, B, D2), jnp.int32)
    out = _mek(x, w_flat, dummy)
    o = jax.lax.bitcast_convert_type(out, jnp.uint32)
    lo_f = jax.lax.bitcast_convert_type(o << 16, jnp.float32)
    hi_f = jax.lax.bitcast_convert_type(o & jnp.uint32(0xFFFF0000), jnp.float32)
    return jnp.concatenate([lo_f, hi_f], axis=1).reshape(N, S, D)


# speedup vs baseline: 2.1477x; 2.1477x over previous
import functools

import jax
import jax.numpy as jnp
from jax import lax
from jax.experimental import pallas as pl
from jax.experimental.pallas import tpu as pltpu
from jax.experimental.pallas import tpu_sc as plsc

L = 8
NT = 1024
D = 1024
N = 4
S = 2048
ROWS = N * S
NC = 2
NS = 16
NW = NC * NS
RPW = ROWS // NW
B = 4
NB = RPW // B
NBUF = 3
LANES = 16


def _fire(w_hbm, idx_v, gath_v, sem, b, buf):
    for l in range(L):
        pltpu.async_copy(
            w_hbm.at[idx_v.at[l, pl.ds(b * B, B)]],
            gath_v.at[buf, l],
            sem,
        )


def _drain(dummy_hbm, gath_v, sem, buf):
    pltpu.make_async_copy(
        dummy_hbm,
        gath_v.at[buf],
        sem,
    ).wait()


def _accum(gath_v, outb_v, buf):
    for j in range(B):
        def cbody(c, _, j=j):
            o = pl.ds(pl.multiple_of(c * LANES, LANES), LANES)
            acc = gath_v[buf, 0, j, o]
            for l in range(1, L):
                acc = acc + gath_v[buf, l, j, o]
            outb_v[j, o] = acc
            return 0
        lax.fori_loop(0, D // LANES, cbody, 0)


def _body(x_hbm, w_hbm, dummy_hbm, out_hbm, idx_v, gath_v, outb_v,
          sem0, sem1, sem2):
    sems = (sem0, sem1, sem2)
    cid = lax.axis_index("c")
    sid = lax.axis_index("s")
    wid = sid * NC + cid
    n = wid // (S // RPW)
    s0 = (wid % (S // RPW)) * RPW
    row0 = wid * RPW

    for l in range(L):
        pltpu.sync_copy(x_hbm.at[n, l, pl.ds(s0, RPW)], idx_v.at[l])

    def off_body(i, _):
        o = pl.ds(pl.multiple_of(i * LANES, LANES), LANES)
        for l in range(1, L):
            idx_v[l, o] = idx_v[l, o] + (l * NT)
        return 0
    lax.fori_loop(0, RPW // LANES, off_body, 0)

    _fire(w_hbm, idx_v, gath_v, sems[0], 0, 0)
    _fire(w_hbm, idx_v, gath_v, sems[1], 1, 1)

    def step(t, _):
        for k in range(NBUF):
            @pl.when(t % NBUF == k)
            def _(k=k):
                _fire(w_hbm, idx_v, gath_v, sems[(k + 2) % NBUF],
                      jnp.minimum(t + 2, NB - 1), (k + 2) % NBUF)
                _drain(dummy_hbm, gath_v, sems[k], k)
                _accum(gath_v, outb_v, k)
                pltpu.sync_copy(outb_v,
                                out_hbm.at[pl.ds(row0 + t * B, B)])
        return 0

    lax.fori_loop(0, NB, step, 0)
    _drain(dummy_hbm, gath_v, sems[NB % NBUF], NB % NBUF)
    _drain(dummy_hbm, gath_v, sems[(NB + 1) % NBUF], (NB + 1) % NBUF)


_mek = functools.partial(
    pl.kernel,
    out_type=jax.ShapeDtypeStruct((ROWS, D), jnp.float32),
    mesh=plsc.VectorSubcoreMesh(core_axis_name="c", subcore_axis_name="s"),
    scratch_types=[
        pltpu.VMEM((L, RPW), jnp.int32),
        pltpu.VMEM((NBUF, L, B, D), jnp.float32),
        pltpu.VMEM((B, D), jnp.float32),
        pltpu.SemaphoreType.DMA,
        pltpu.SemaphoreType.DMA,
        pltpu.SemaphoreType.DMA,
    ],
)(_body)


@jax.jit
def kernel(x, weight):
    x = x.astype(jnp.int32)
    w_flat = weight.reshape(L * NT, D)
    dummy = jnp.zeros((L, B, D), jnp.float32)
    out = _mek(x, w_flat, dummy)
    return out.reshape(N, S, D)
